# baseline (speedup 1.0000x reference)
import jax
import jax.numpy as jnp
from jax import lax
from jax.experimental import pallas as pl
from jax.experimental.pallas import tpu as pltpu
from jax.experimental.pallas import tpu_sc as plsc

_TOPK = 512
_INT_MIN = -(2**31)
_N = 32768
_B = 1024
_NW = 32
_RPW = _B // _NW
_NB = 256


def _i32(v):
    return jnp.int32(v)


def _sc_body(x_hbm, o_hbm, buf0, buf1, hist, totbuf, cumbuf, pbuf, si0, si1, so0, so1):
    cid = lax.axis_index("c")
    sid = lax.axis_index("s")
    wid = sid * 2 + cid
    base_row = wid * _RPW
    lane = lax.iota(jnp.int32, 16)
    zeros16 = jnp.zeros((16,), jnp.int32)
    ones16 = jnp.ones((16,), jnp.int32)
    lane_base = lane * _NB

    @plsc.parallel_loop(0, (_NB * 16) // 16, unroll=8)
    def _(i):
        hist[pl.ds(i * 16, 16)] = zeros16

    def scan_level(n_l, k_l):
        @plsc.parallel_loop(0, _NB // 16, unroll=2)
        def _(ci):
            acc = zeros16
            for l in range(16):
                off = l * _NB + ci * 16
                acc = acc + hist[pl.ds(off, 16)]
                hist[pl.ds(off, 16)] = zeros16
            totbuf[pl.ds(ci * 16, 16)] = acc
            cumbuf[pl.ds(ci * 16, 16)] = lax.cumsum(acc, axis=0)

        ct = plsc.load_gather(cumbuf, [lane * 16 + 15])
        cum_ct = lax.cumsum(ct, axis=0)
        pbuf[pl.ds(0, 16)] = cum_ct - ct

        @plsc.parallel_loop(0, _NB // 16, unroll=4, carry=zeros16)
        def cnt(ci, acc):
            t = totbuf[pl.ds(ci * 16, 16)]
            pfx = plsc.load_gather(pbuf, [jnp.broadcast_to(ci, (16,)).astype(jnp.int32)])
            cm = cumbuf[pl.ds(ci * 16, 16)] + pfx
            cond = (n_l - cm + t) >= k_l
            return acc + plsc.all_reduce_population_count(cond)

        bstar = cnt - 1
        cumb = plsc.load_gather(cumbuf, [bstar]) + plsc.load_gather(
            pbuf, [lax.shift_right_logical(bstar, 4)]
        )
        totb = plsc.load_gather(totbuf, [bstar])
        return bstar, cumb, totb

    def process_row(buf, mid_fn):
        @plsc.parallel_loop(0, _N // 16, unroll=8)
        def _(i):
            x = buf[pl.ds(i * 16, 16)]
            bits = lax.bitcast_convert_type(x, jnp.int32)
            u = jnp.where(bits < 0, bits ^ _i32(-1), bits ^ _i32(_INT_MIN))
            buf[pl.ds(i * 16, 16)] = lax.bitcast_convert_type(u, jnp.float32)
            b1v = lax.shift_right_logical(u, 24)
            plsc.addupdate_scatter(hist, [lane_base + b1v], ones16)

        mid_fn()

        n1 = jnp.broadcast_to(_i32(_N), (16,))
        k1 = jnp.broadcast_to(_i32(_TOPK), (16,))
        b1, cumb1, totb1 = scan_level(n1, k1)
        n2 = totb1
        k2 = k1 - (n1 - cumb1)

        @plsc.parallel_loop(0, _N // 16, unroll=8)
        def _(i):
            u = lax.bitcast_convert_type(buf[pl.ds(i * 16, 16)], jnp.int32)
            m = lax.shift_right_logical(u, 24) == b1
            b2v = lax.shift_right_logical(u, 16) & _i32(0xFF)
            plsc.addupdate_scatter(hist, [lane_base + b2v], ones16, mask=m)

        b2, cumb2, totb2 = scan_level(n2, k2)
        n3 = totb2
        k3 = k2 - (n2 - cumb2)
        pref16 = (b1 << 8) | b2

        @plsc.parallel_loop(0, _N // 16, unroll=8)
        def _(i):
            u = lax.bitcast_convert_type(buf[pl.ds(i * 16, 16)], jnp.int32)
            m = lax.shift_right_logical(u, 16) == pref16
            b3v = lax.shift_right_logical(u, 8) & _i32(0xFF)
            plsc.addupdate_scatter(hist, [lane_base + b3v], ones16, mask=m)

        b3, cumb3, totb3 = scan_level(n3, k3)
        n4 = totb3
        k4 = k3 - (n3 - cumb3)
        pref24 = (pref16 << 8) | b3

        @plsc.parallel_loop(0, _N // 16, unroll=8)
        def _(i):
            u = lax.bitcast_convert_type(buf[pl.ds(i * 16, 16)], jnp.int32)
            m = lax.shift_right_logical(u, 8) == pref24
            b4v = u & _i32(0xFF)
            plsc.addupdate_scatter(hist, [lane_base + b4v], ones16, mask=m)

        b4, _, _ = scan_level(n4, k4)

        kth_u = (b1 << 24) | (b2 << 16) | (b3 << 8) | b4
        sthr = jnp.maximum(kth_u ^ _i32(_INT_MIN), 0)

        @plsc.parallel_loop(0, _N // 16, unroll=8)
        def _(i):
            u = lax.bitcast_convert_type(buf[pl.ds(i * 16, 16)], jnp.int32)
            s = u ^ _i32(_INT_MIN)
            o = jnp.where(s >= sthr, s, 0)
            buf[pl.ds(i * 16, 16)] = lax.bitcast_convert_type(o, jnp.float32)

    npairs = _RPW // 2
    pltpu.async_copy(x_hbm.at[base_row], buf0, si0)
    pltpu.async_copy(x_hbm.at[base_row + 1], buf1, si1)

    def pair_body(t, carry):
        a = base_row + 2 * t
        b = a + 1

        pltpu.make_async_copy(x_hbm.at[a], buf0, si0).wait()

        def mid_a():
            @pl.when(t > 0)
            def _():
                pltpu.make_async_copy(buf1, o_hbm.at[b - 2], so1).wait()
                pltpu.async_copy(x_hbm.at[b], buf1, si1)

        process_row(buf0, mid_a)
        pltpu.async_copy(buf0, o_hbm.at[a], so0)

        pltpu.make_async_copy(x_hbm.at[b], buf1, si1).wait()

        def mid_b():
            pltpu.make_async_copy(buf0, o_hbm.at[a], so0).wait()

            @pl.when(t < npairs - 1)
            def _():
                pltpu.async_copy(x_hbm.at[a + 2], buf0, si0)

        process_row(buf1, mid_b)
        pltpu.async_copy(buf1, o_hbm.at[b], so1)
        return carry

    lax.fori_loop(0, npairs, pair_body, 0)
    pltpu.make_async_copy(buf1, o_hbm.at[base_row + _RPW - 1], so1).wait()


def kernel(features):
    b, l, d = features.shape
    flat = features.reshape(b, l * d)
    mesh = plsc.VectorSubcoreMesh(core_axis_name="c", subcore_axis_name="s")
    out = pl.kernel(
        _sc_body,
        out_type=jax.ShapeDtypeStruct((b, l * d), jnp.float32),
        mesh=mesh,
        compiler_params=pltpu.CompilerParams(needs_layout_passes=False),
        scratch_types=[
            pltpu.VMEM((_N,), jnp.float32),
            pltpu.VMEM((_N,), jnp.float32),
            pltpu.VMEM((_NB * 16,), jnp.int32),
            pltpu.VMEM((_NB,), jnp.int32),
            pltpu.VMEM((_NB,), jnp.int32),
            pltpu.VMEM((16,), jnp.int32),
            pltpu.SemaphoreType.DMA,
            pltpu.SemaphoreType.DMA,
            pltpu.SemaphoreType.DMA,
            pltpu.SemaphoreType.DMA,
        ],
    )(flat)
    return out.reshape(b, l, d)


# speedup vs baseline: 1.1336x; 1.1336x over previous
import jax
import jax.numpy as jnp
from jax import lax
from jax.experimental import pallas as pl
from jax.experimental.pallas import tpu as pltpu
from jax.experimental.pallas import tpu_sc as plsc

_TOPK = 512
_INT_MIN = -(2**31)
_N = 32768
_B = 1024
_NW = 32
_RPW = _B // _NW
_NB = 256


def _i32(v):
    return jnp.int32(v)


def _sc_body(
    x_hbm, o_hbm, buf0, buf1, cand, hist, totbuf, cumbuf, pbuf, si0, si1, so0, so1
):
    cid = lax.axis_index("c")
    sid = lax.axis_index("s")
    wid = sid * 2 + cid
    base_row = wid * _RPW
    lane = lax.iota(jnp.int32, 16)
    zeros16 = jnp.zeros((16,), jnp.int32)
    ones16 = jnp.ones((16,), jnp.int32)
    lane_base = lane * _NB

    @plsc.parallel_loop(0, (_NB * 16) // 16, unroll=8)
    def _(i):
        hist[pl.ds(i * 16, 16)] = zeros16

    def scan_level(n_l, k_l):
        @plsc.parallel_loop(0, _NB // 16, unroll=2)
        def _(ci):
            acc = zeros16
            for l in range(16):
                off = l * _NB + ci * 16
                acc = acc + hist[pl.ds(off, 16)]
                hist[pl.ds(off, 16)] = zeros16
            totbuf[pl.ds(ci * 16, 16)] = acc
            cumbuf[pl.ds(ci * 16, 16)] = lax.cumsum(acc, axis=0)

        ct = plsc.load_gather(cumbuf, [lane * 16 + 15])
        cum_ct = lax.cumsum(ct, axis=0)
        pbuf[pl.ds(0, 16)] = cum_ct - ct

        @plsc.parallel_loop(0, _NB // 16, unroll=4, carry=zeros16)
        def cnt(ci, acc):
            t = totbuf[pl.ds(ci * 16, 16)]
            pfx = plsc.load_gather(pbuf, [jnp.broadcast_to(ci, (16,)).astype(jnp.int32)])
            cm = cumbuf[pl.ds(ci * 16, 16)] + pfx
            cond = (n_l - cm + t) >= k_l
            return acc + plsc.all_reduce_population_count(cond)

        bstar = cnt - 1
        cumb = plsc.load_gather(cumbuf, [bstar]) + plsc.load_gather(
            pbuf, [lax.shift_right_logical(bstar, 4)]
        )
        totb = plsc.load_gather(totbuf, [bstar])
        return bstar, cumb, totb

    def cand_hist(ncand, shift, prefix, pshift):
        trips = lax.div(jnp.max(ncand, axis=0) + 127, _i32(128))

        @plsc.parallel_loop(0, trips, unroll=1)
        def _(i):
            base = i * 128
            for j in range(8):
                eidx = base + j * 16 + lane
                u = plsc.load_gather(cand, [eidx])
                m = (eidx < ncand) & (lax.shift_right_logical(u, pshift) == prefix)
                bv = lax.shift_right_logical(u, shift) & _i32(0xFF)
                plsc.addupdate_scatter(hist, [lane_base + bv], ones16, mask=m)

    def process_row(buf, mid_fn):
        @plsc.parallel_loop(0, _N // 16, unroll=8)
        def _(i):
            x = buf[pl.ds(i * 16, 16)]
            bits = lax.bitcast_convert_type(x, jnp.int32)
            u = jnp.where(bits < 0, bits ^ _i32(-1), bits ^ _i32(_INT_MIN))
            buf[pl.ds(i * 16, 16)] = lax.bitcast_convert_type(u, jnp.float32)
            b1v = lax.shift_right_logical(u, 24)
            plsc.addupdate_scatter(hist, [lane_base + b1v], ones16)

        mid_fn()

        n1 = jnp.broadcast_to(_i32(_N), (16,))
        k1 = jnp.broadcast_to(_i32(_TOPK), (16,))
        b1, cumb1, totb1 = scan_level(n1, k1)
        n2 = totb1
        k2 = k1 - (n1 - cumb1)

        @plsc.parallel_loop(0, _N // 16, unroll=8, carry=zeros16)
        def off(i, acc):
            u = lax.bitcast_convert_type(buf[pl.ds(i * 16, 16)], jnp.int32)
            m = lax.shift_right_logical(u, 24) == b1
            pos = acc + lax.cumsum(m.astype(jnp.int32), axis=0) - 1
            plsc.store_scatter(cand, [pos], u, mask=m)
            return acc + plsc.all_reduce_population_count(m)

        del off
        cand_hist(n2, 16, b1, 24)
        b2, cumb2, totb2 = scan_level(n2, k2)
        n3 = totb2
        k3 = k2 - (n2 - cumb2)
        pref16 = (b1 << 8) | b2

        cand_hist(n2, 8, pref16, 16)
        b3, cumb3, totb3 = scan_level(n3, k3)
        n4 = totb3
        k4 = k3 - (n3 - cumb3)
        pref24 = (pref16 << 8) | b3

        cand_hist(n2, 0, pref24, 8)
        b4, _, _ = scan_level(n4, k4)

        kth_u = (b1 << 24) | (b2 << 16) | (b3 << 8) | b4
        sthr = jnp.maximum(kth_u ^ _i32(_INT_MIN), 0)

        @plsc.parallel_loop(0, _N // 16, unroll=8)
        def _(i):
            u = lax.bitcast_convert_type(buf[pl.ds(i * 16, 16)], jnp.int32)
            s = u ^ _i32(_INT_MIN)
            o = jnp.where(s >= sthr, s, 0)
            buf[pl.ds(i * 16, 16)] = lax.bitcast_convert_type(o, jnp.float32)

    npairs = _RPW // 2
    pltpu.async_copy(x_hbm.at[base_row], buf0, si0)
    pltpu.async_copy(x_hbm.at[base_row + 1], buf1, si1)

    def pair_body(t, carry):
        a = base_row + 2 * t
        b = a + 1

        pltpu.make_async_copy(x_hbm.at[a], buf0, si0).wait()

        def mid_a():
            @pl.when(t > 0)
            def _():
                pltpu.make_async_copy(buf1, o_hbm.at[b - 2], so1).wait()
                pltpu.async_copy(x_hbm.at[b], buf1, si1)

        process_row(buf0, mid_a)
        pltpu.async_copy(buf0, o_hbm.at[a], so0)

        pltpu.make_async_copy(x_hbm.at[b], buf1, si1).wait()

        def mid_b():
            pltpu.make_async_copy(buf0, o_hbm.at[a], so0).wait()

            @pl.when(t < npairs - 1)
            def _():
                pltpu.async_copy(x_hbm.at[a + 2], buf0, si0)

        process_row(buf1, mid_b)
        pltpu.async_copy(buf1, o_hbm.at[b], so1)
        return carry

    lax.fori_loop(0, npairs, pair_body, 0)
    pltpu.make_async_copy(buf1, o_hbm.at[base_row + _RPW - 1], so1).wait()


def kernel(features):
    b, l, d = features.shape
    flat = features.reshape(b, l * d)
    mesh = plsc.VectorSubcoreMesh(core_axis_name="c", subcore_axis_name="s")
    out = pl.kernel(
        _sc_body,
        out_type=jax.ShapeDtypeStruct((b, l * d), jnp.float32),
        mesh=mesh,
        compiler_params=pltpu.CompilerParams(needs_layout_passes=False),
        scratch_types=[
            pltpu.VMEM((_N,), jnp.float32),
            pltpu.VMEM((_N,), jnp.float32),
            pltpu.VMEM((_N + 128,), jnp.int32),
            pltpu.VMEM((_NB * 16,), jnp.int32),
            pltpu.VMEM((_NB,), jnp.int32),
            pltpu.VMEM((_NB,), jnp.int32),
            pltpu.VMEM((16,), jnp.int32),
            pltpu.SemaphoreType.DMA,
            pltpu.SemaphoreType.DMA,
            pltpu.SemaphoreType.DMA,
            pltpu.SemaphoreType.DMA,
        ],
    )(flat)
    return out.reshape(b, l, d)
